# baseline (speedup 1.0000x reference)
import functools

import jax
import jax.numpy as jnp
from jax import lax
from jax.experimental import pallas as pl
from jax.experimental.pallas import tpu as pltpu
from jax.experimental.pallas import tpu_sc as plsc

B, L, DIM = 1024, 200, 128
DAY_SIZE, TIME_SIZE = 32, 64
OUT_D = DIM + DAY_SIZE + TIME_SIZE
N = B * L

_info = plsc.get_sparse_core_info()
NC, NS, LANES = _info.num_cores, _info.num_subcores, _info.num_lanes
NW = NC * NS
ROWS_PER_W = N // NW
CHUNK = 128
NCHUNK = ROWS_PER_W // CHUNK

_mesh = plsc.VectorSubcoreMesh(core_axis_name="c", subcore_axis_name="s")


@functools.partial(
    pl.kernel,
    out_type=jax.ShapeDtypeStruct((N, OUT_D), jnp.float32),
    mesh=_mesh,
    compiler_params=pltpu.CompilerParams(use_tc_tiling_on_sc=False,
                                          needs_layout_passes=False),
    scratch_types=[
        pltpu.VMEM((2 * CHUNK,), jnp.int32),
        pltpu.VMEM((CHUNK,), jnp.int32),
        pltpu.VMEM((CHUNK,), jnp.int32),
        pltpu.VMEM((CHUNK, DIM), jnp.float32),
        pltpu.VMEM((CHUNK, DAY_SIZE), jnp.float32),
        pltpu.VMEM((CHUNK, TIME_SIZE), jnp.float32),
        pltpu.SemaphoreType.DMA,
    ],
)
def _sc_body(inp_hbm, idx_hbm, day_hbm, time_hbm, out_hbm,
             idxraw_v, d_v, t_v, inp_v, day_v, time_v, sem):
    wid = lax.axis_index("s") * NC + lax.axis_index("c")
    base = wid * ROWS_PER_W

    @pl.loop(0, NCHUNK)
    def _chunk(g):
        r0 = base + g * CHUNK

        c_idx = pltpu.async_copy(idx_hbm.at[pl.ds(2 * r0, 2 * CHUNK)],
                                 idxraw_v, sem)
        c_inp = pltpu.async_copy(inp_hbm.at[pl.ds(r0, CHUNK)], inp_v, sem)
        c_idx.wait()

        lanes = lax.iota(jnp.int32, LANES)
        for j in range(CHUNK // LANES):
            off = 2 * LANES * j
            dvec = plsc.load_gather(idxraw_v, [off + 2 * lanes])
            tvec = plsc.load_gather(idxraw_v, [off + 2 * lanes + 1])
            d_v[pl.ds(j * LANES, LANES)] = dvec
            t_v[pl.ds(j * LANES, LANES)] = tvec

        g_day = pltpu.async_copy(day_hbm.at[d_v], day_v, sem)
        g_time = pltpu.async_copy(time_hbm.at[t_v], time_v, sem)
        g_day.wait()
        g_time.wait()
        c_inp.wait()

        o_inp = pltpu.async_copy(
            inp_v, out_hbm.at[pl.ds(r0, CHUNK), pl.ds(0, DIM)], sem)
        o_day = pltpu.async_copy(
            day_v, out_hbm.at[pl.ds(r0, CHUNK), pl.ds(DIM, DAY_SIZE)], sem)
        o_time = pltpu.async_copy(
            time_v,
            out_hbm.at[pl.ds(r0, CHUNK), pl.ds(DIM + DAY_SIZE, TIME_SIZE)],
            sem)
        o_inp.wait()
        o_day.wait()
        o_time.wait()


def kernel(inp, daytime, emb_day, emb_time):
    inp2 = inp.reshape(N, DIM)
    idx = daytime.astype(jnp.int32).reshape(2 * N)
    out = _sc_body(inp2, idx, emb_day, emb_time)
    return out.reshape(B, L, OUT_D)


# speedup vs baseline: 1.0002x; 1.0002x over previous
import functools

import jax
import jax.numpy as jnp
from jax import lax
from jax.experimental import pallas as pl
from jax.experimental.pallas import tpu as pltpu
from jax.experimental.pallas import tpu_sc as plsc

B, L, DIM = 1024, 200, 128
DAY_SIZE, TIME_SIZE = 32, 64
OUT_D = DIM + DAY_SIZE + TIME_SIZE
N = B * L

_info = plsc.get_sparse_core_info()
NC, NS, LANES = _info.num_cores, _info.num_subcores, _info.num_lanes
NW = NC * NS
ROWS_PER_W = N // NW
CHUNK = 80
NCHUNK = ROWS_PER_W // CHUNK
NBUF = 4
NOUTER = NCHUNK // NBUF

_mesh = plsc.VectorSubcoreMesh(core_axis_name="c", subcore_axis_name="s")


@functools.partial(
    pl.kernel,
    out_type=jax.ShapeDtypeStruct((N, OUT_D), jnp.float32),
    mesh=_mesh,
    compiler_params=pltpu.CompilerParams(use_tc_tiling_on_sc=False,
                                         needs_layout_passes=False),
    scratch_types=(
        [pltpu.VMEM((NBUF, 2 * CHUNK), jnp.int32)]
        + [pltpu.VMEM((NBUF, CHUNK), jnp.int32)] * 2
        + [pltpu.VMEM((NBUF, CHUNK, DIM), jnp.float32)]
        + [pltpu.VMEM((NBUF, CHUNK, DAY_SIZE), jnp.float32)]
        + [pltpu.VMEM((NBUF, CHUNK, TIME_SIZE), jnp.float32)]
        + [pltpu.SemaphoreType.DMA] * (3 * NBUF)
    ),
)
def _sc_body(inp_hbm, idx_hbm, day_hbm, time_hbm, out_hbm,
             idxraw_v, d_v, t_v, inp_v, day_v, time_v, *sems):
    in_sem = sems[0:NBUF]
    g_sem = sems[NBUF:2 * NBUF]
    out_sem = sems[2 * NBUF:3 * NBUF]

    wid = lax.axis_index("s") * NC + lax.axis_index("c")
    base = wid * ROWS_PER_W

    def fire_in(g, b):
        r0 = base + g * CHUNK
        pltpu.async_copy(idx_hbm.at[pl.ds(2 * r0, 2 * CHUNK)],
                         idxraw_v.at[b], in_sem[b])
        pltpu.async_copy(inp_hbm.at[pl.ds(r0, CHUNK)], inp_v.at[b], in_sem[b])

    def wait_in(b):
        pltpu.make_async_copy(idx_hbm.at[pl.ds(0, 2 * CHUNK)],
                              idxraw_v.at[b], in_sem[b]).wait()
        pltpu.make_async_copy(inp_hbm.at[pl.ds(0, CHUNK)],
                              inp_v.at[b], in_sem[b]).wait()

    def fire_out(g, b):
        r0 = base + g * CHUNK
        pltpu.async_copy(
            inp_v.at[b], out_hbm.at[pl.ds(r0, CHUNK), pl.ds(0, DIM)],
            out_sem[b])
        pltpu.async_copy(
            day_v.at[b], out_hbm.at[pl.ds(r0, CHUNK), pl.ds(DIM, DAY_SIZE)],
            out_sem[b])
        pltpu.async_copy(
            time_v.at[b],
            out_hbm.at[pl.ds(r0, CHUNK), pl.ds(DIM + DAY_SIZE, TIME_SIZE)],
            out_sem[b])

    def wait_out(b):
        pltpu.make_async_copy(
            inp_v.at[b], out_hbm.at[pl.ds(0, CHUNK), pl.ds(0, DIM)],
            out_sem[b]).wait()
        pltpu.make_async_copy(
            day_v.at[b], out_hbm.at[pl.ds(0, CHUNK), pl.ds(DIM, DAY_SIZE)],
            out_sem[b]).wait()
        pltpu.make_async_copy(
            time_v.at[b],
            out_hbm.at[pl.ds(0, CHUNK), pl.ds(DIM + DAY_SIZE, TIME_SIZE)],
            out_sem[b]).wait()

    for g0 in range(NBUF - 1):
        fire_in(g0, g0)

    lanes = lax.iota(jnp.int32, LANES)

    @pl.loop(0, NOUTER)
    def _blk(k):
        for j in range(NBUF):
            g = k * NBUF + j
            b = j

            wait_in(b)

            for q in range(CHUNK // LANES):
                off = 2 * LANES * q
                dvec = plsc.load_gather(idxraw_v.at[b], [off + 2 * lanes])
                tvec = plsc.load_gather(idxraw_v.at[b], [off + 2 * lanes + 1])
                d_v[b, pl.ds(q * LANES, LANES)] = dvec
                t_v[b, pl.ds(q * LANES, LANES)] = tvec

            pltpu.async_copy(day_hbm.at[d_v.at[b]], day_v.at[b], g_sem[b])
            pltpu.async_copy(time_hbm.at[t_v.at[b]], time_v.at[b], g_sem[b])

            f = g + NBUF - 1
            fb = (j + NBUF - 1) % NBUF

            @pl.when(f < NCHUNK)
            def _():
                @pl.when(g >= 1)
                def _():
                    wait_out(fb)
                fire_in(f, fb)

            pltpu.make_async_copy(day_hbm.at[d_v.at[b]], day_v.at[b],
                                  g_sem[b]).wait()
            pltpu.make_async_copy(time_hbm.at[t_v.at[b]], time_v.at[b],
                                  g_sem[b]).wait()

            fire_out(g, b)

    for g in range(NCHUNK - NBUF, NCHUNK):
        wait_out(g % NBUF)


def kernel(inp, daytime, emb_day, emb_time):
    inp2 = inp.reshape(N, DIM)
    idx = daytime.astype(jnp.int32).reshape(2 * N)
    out = _sc_body(inp2, idx, emb_day, emb_time)
    return out.reshape(B, L, OUT_D)


# speedup vs baseline: 2.5798x; 2.5794x over previous
import functools

import jax
import jax.numpy as jnp
from jax import lax
from jax.experimental import pallas as pl
from jax.experimental.pallas import tpu as pltpu
from jax.experimental.pallas import tpu_sc as plsc

B, L, DIM = 1024, 200, 128
DAY_SIZE, TIME_SIZE = 32, 64
NUM_DAYS, DAILY_TIMES = 7, 288
OUT_D = DIM + DAY_SIZE + TIME_SIZE
N = B * L

_info = plsc.get_sparse_core_info()
NC, NS, LANES = _info.num_cores, _info.num_subcores, _info.num_lanes
NW = NC * NS
ROWS_PER_W = N // NW
CHUNK = 80
NCHUNK = ROWS_PER_W // CHUNK
NBUF = 4
NOUTER = NCHUNK // NBUF

_mesh = plsc.VectorSubcoreMesh(core_axis_name="c", subcore_axis_name="s")


@functools.partial(
    pl.kernel,
    out_type=jax.ShapeDtypeStruct((N, OUT_D), jnp.float32),
    mesh=_mesh,
    compiler_params=pltpu.CompilerParams(use_tc_tiling_on_sc=False,
                                         needs_layout_passes=False),
    scratch_types=(
        [pltpu.VMEM((NBUF, 2 * CHUNK + LANES), jnp.int32)]
        + [pltpu.VMEM((NBUF, CHUNK, DIM), jnp.float32)]
        + [pltpu.VMEM((NBUF, CHUNK, DAY_SIZE), jnp.float32)]
        + [pltpu.VMEM((NBUF, CHUNK, TIME_SIZE), jnp.float32)]
        + [pltpu.VMEM((NUM_DAYS * DAY_SIZE,), jnp.float32)]
        + [pltpu.VMEM((DAILY_TIMES * TIME_SIZE,), jnp.float32)]
        + [pltpu.SemaphoreType.DMA] * (2 * NBUF)
    ),
)
def _sc_body(inp_hbm, idx_hbm, day_hbm, time_hbm, out_hbm,
             idxraw_v, inp_v, day_v, time_v, day_tab, time_tab, *sems):
    in_sem = sems[0:NBUF]
    out_sem = sems[NBUF:2 * NBUF]

    wid = lax.axis_index("s") * NC + lax.axis_index("c")
    base = wid * ROWS_PER_W

    def fire_in(g, b):
        r0 = base + g * CHUNK
        pltpu.async_copy(idx_hbm.at[pl.ds(2 * r0, 2 * CHUNK)],
                         idxraw_v.at[b, pl.ds(0, 2 * CHUNK)], in_sem[b])
        pltpu.async_copy(inp_hbm.at[pl.ds(r0, CHUNK)], inp_v.at[b], in_sem[b])

    def wait_in(b):
        pltpu.make_async_copy(idx_hbm.at[pl.ds(0, 2 * CHUNK)],
                              idxraw_v.at[b, pl.ds(0, 2 * CHUNK)],
                              in_sem[b]).wait()
        pltpu.make_async_copy(inp_hbm.at[pl.ds(0, CHUNK)],
                              inp_v.at[b], in_sem[b]).wait()

    def fire_out(g, b):
        r0 = base + g * CHUNK
        pltpu.async_copy(
            inp_v.at[b], out_hbm.at[pl.ds(r0, CHUNK), pl.ds(0, DIM)],
            out_sem[b])
        pltpu.async_copy(
            day_v.at[b], out_hbm.at[pl.ds(r0, CHUNK), pl.ds(DIM, DAY_SIZE)],
            out_sem[b])
        pltpu.async_copy(
            time_v.at[b],
            out_hbm.at[pl.ds(r0, CHUNK), pl.ds(DIM + DAY_SIZE, TIME_SIZE)],
            out_sem[b])

    def wait_out(b):
        pltpu.make_async_copy(
            inp_v.at[b], out_hbm.at[pl.ds(0, CHUNK), pl.ds(0, DIM)],
            out_sem[b]).wait()
        pltpu.make_async_copy(
            day_v.at[b], out_hbm.at[pl.ds(0, CHUNK), pl.ds(DIM, DAY_SIZE)],
            out_sem[b]).wait()
        pltpu.make_async_copy(
            time_v.at[b],
            out_hbm.at[pl.ds(0, CHUNK), pl.ds(DIM + DAY_SIZE, TIME_SIZE)],
            out_sem[b]).wait()

    pltpu.sync_copy(day_hbm, day_tab)
    pltpu.sync_copy(time_hbm, time_tab)

    for g0 in range(NBUF - 1):
        fire_in(g0, g0)

    @pl.loop(0, NOUTER)
    def _blk(k):
        for j in range(NBUF):
            g = k * NBUF + j
            b = j

            wait_in(b)

            f = g + NBUF - 1
            fb = (j + NBUF - 1) % NBUF

            @pl.when(f < NCHUNK)
            def _():
                @pl.when(g >= 1)
                def _():
                    wait_out(fb)
                fire_in(f, fb)

            @pl.loop(0, CHUNK, unroll=8)
            def _row(r):
                pair = idxraw_v[b, pl.ds(2 * r, LANES)]
                do = DAY_SIZE * pair[0]
                to = TIME_SIZE * pair[1]
                for c in range(0, DAY_SIZE, LANES):
                    day_v[b, r, pl.ds(c, LANES)] = day_tab[pl.ds(do + c,
                                                                 LANES)]
                for c in range(0, TIME_SIZE, LANES):
                    time_v[b, r, pl.ds(c, LANES)] = time_tab[pl.ds(to + c,
                                                                   LANES)]

            fire_out(g, b)

    for g in range(NCHUNK - NBUF, NCHUNK):
        wait_out(g % NBUF)


def kernel(inp, daytime, emb_day, emb_time):
    inp2 = inp.reshape(N, DIM)
    idx = daytime.astype(jnp.int32).reshape(2 * N)
    out = _sc_body(inp2, idx,
                   emb_day.reshape(NUM_DAYS * DAY_SIZE),
                   emb_time.reshape(DAILY_TIMES * TIME_SIZE))
    return out.reshape(B, L, OUT_D)


# speedup vs baseline: 3.4994x; 1.3564x over previous
import functools

import jax
import jax.numpy as jnp
from jax import lax
from jax.experimental import pallas as pl
from jax.experimental.pallas import tpu as pltpu
from jax.experimental.pallas import tpu_sc as plsc

B, L, DIM = 1024, 200, 128
DAY_SIZE, TIME_SIZE = 32, 64
NUM_DAYS, DAILY_TIMES = 7, 288
DT = DAY_SIZE + TIME_SIZE
OUT_D = DIM + DT
N = B * L

_info = plsc.get_sparse_core_info()
NC, NS, LANES = _info.num_cores, _info.num_subcores, _info.num_lanes
NW = NC * NS
ROWS_PER_W = N // NW
CHUNK = 64
NCHUNK = ROWS_PER_W // CHUNK
NBUF = 4
NOUTER = NCHUNK // NBUF

_mesh = plsc.VectorSubcoreMesh(core_axis_name="c", subcore_axis_name="s")


@functools.partial(
    pl.kernel,
    out_type=jax.ShapeDtypeStruct((N, OUT_D), jnp.float32),
    mesh=_mesh,
    compiler_params=pltpu.CompilerParams(use_tc_tiling_on_sc=True),
    scratch_types=(
        [pltpu.VMEM((2 * CHUNK + LANES,), jnp.int32)] * NBUF
        + [pltpu.VMEM((CHUNK, DIM), jnp.float32)] * NBUF
        + [pltpu.VMEM((CHUNK, DT), jnp.float32)] * NBUF
        + [pltpu.VMEM((NUM_DAYS * DAY_SIZE,), jnp.float32)]
        + [pltpu.VMEM((DAILY_TIMES * TIME_SIZE,), jnp.float32)]
        + [pltpu.SemaphoreType.DMA] * (2 * NBUF)
    ),
)
def _sc_body(inp_hbm, idx_hbm, day_hbm, time_hbm, out_hbm, *scratch):
    idxraw_v = scratch[0:NBUF]
    inp_v = scratch[NBUF:2 * NBUF]
    dt_v = scratch[2 * NBUF:3 * NBUF]
    day_tab = scratch[3 * NBUF]
    time_tab = scratch[3 * NBUF + 1]
    in_sem = scratch[3 * NBUF + 2:3 * NBUF + 2 + NBUF]
    out_sem = scratch[3 * NBUF + 2 + NBUF:3 * NBUF + 2 + 2 * NBUF]

    wid = lax.axis_index("s") * NC + lax.axis_index("c")
    base = wid * ROWS_PER_W

    def fire_in(g, b):
        r0 = base + g * CHUNK
        pltpu.async_copy(idx_hbm.at[pl.ds(2 * r0, 2 * CHUNK)],
                         idxraw_v[b].at[pl.ds(0, 2 * CHUNK)], in_sem[b])
        pltpu.async_copy(inp_hbm.at[pl.ds(r0, CHUNK)], inp_v[b], in_sem[b])

    def wait_in(b):
        pltpu.make_async_copy(idx_hbm.at[pl.ds(0, 2 * CHUNK)],
                              idxraw_v[b].at[pl.ds(0, 2 * CHUNK)],
                              in_sem[b]).wait()
        pltpu.make_async_copy(inp_hbm.at[pl.ds(0, CHUNK)],
                              inp_v[b], in_sem[b]).wait()

    def fire_out(g, b):
        r0 = base + g * CHUNK
        pltpu.async_copy(
            inp_v[b], out_hbm.at[pl.ds(r0, CHUNK), pl.ds(0, DIM)],
            out_sem[b])
        pltpu.async_copy(
            dt_v[b], out_hbm.at[pl.ds(r0, CHUNK), pl.ds(DIM, DT)],
            out_sem[b])

    def wait_out(b):
        pltpu.make_async_copy(
            inp_v[b], out_hbm.at[pl.ds(0, CHUNK), pl.ds(0, DIM)],
            out_sem[b]).wait()
        pltpu.make_async_copy(
            dt_v[b], out_hbm.at[pl.ds(0, CHUNK), pl.ds(DIM, DT)],
            out_sem[b]).wait()

    pltpu.sync_copy(day_hbm, day_tab)
    pltpu.sync_copy(time_hbm, time_tab)

    for g0 in range(NBUF - 1):
        fire_in(g0, g0)

    @pl.loop(0, NOUTER)
    def _blk(k):
        for j in range(NBUF):
            g = k * NBUF + j
            b = j

            wait_in(b)

            f = g + NBUF - 1
            fb = (j + NBUF - 1) % NBUF

            @pl.when(f < NCHUNK)
            def _():
                @pl.when(g >= 1)
                def _():
                    wait_out(fb)
                fire_in(f, fb)

            @pl.loop(0, CHUNK, unroll=8)
            def _row(r):
                pair = idxraw_v[b][pl.ds(2 * r, LANES)]
                do = DAY_SIZE * pair[0]
                to = TIME_SIZE * pair[1]
                for c in range(0, DAY_SIZE, LANES):
                    dt_v[b][r, pl.ds(c, LANES)] = day_tab[pl.ds(do + c,
                                                                LANES)]
                for c in range(0, TIME_SIZE, LANES):
                    dt_v[b][r, pl.ds(DAY_SIZE + c, LANES)] = time_tab[
                        pl.ds(to + c, LANES)]

            fire_out(g, b)

    for g in range(NCHUNK - NBUF, NCHUNK):
        wait_out(g % NBUF)


def kernel(inp, daytime, emb_day, emb_time):
    inp2 = inp.reshape(N, DIM)
    idx = daytime.astype(jnp.int32).reshape(2 * N)
    out = _sc_body(inp2, idx,
                   emb_day.reshape(NUM_DAYS * DAY_SIZE),
                   emb_time.reshape(DAILY_TIMES * TIME_SIZE))
    return out.reshape(B, L, OUT_D)
